# baseline (speedup 1.0000x reference)
import functools
import jax
import jax.numpy as jnp
from jax import lax
from jax.experimental import pallas as pl
from jax.experimental.pallas import tpu as pltpu
from jax.experimental.pallas import tpu_sc as plsc

NC = 2
NS = 16
NW = NC * NS
LANES = 16
BATCH = 16384
EMB = 64
PADW = 128
BPW = BATCH // NW
CHUNK = 128
NCHUNK = BPW // CHUNK
GROUPS = CHUNK // LANES
TBLK = 2048
SPLIT_M = 128
SPLIT = NW * PADW * SPLIT_M


def _transpose_body(in_ref, eye_ref, out_ref):
    x = in_ref[...]
    t_full = lax.dot_general(x, eye_ref[...],
                             (((0,), (0,)), ((), ())),
                             preferred_element_type=jnp.float32)
    for t in range(TBLK // PADW):
        osl = pl.ds(t * EMB, EMB)
        out_ref[osl, pl.ds(0, EMB)] = t_full[t * PADW:t * PADW + EMB, :]
        out_ref[osl, pl.ds(EMB, EMB)] = (
            t_full[t * PADW + EMB:t * PADW + 2 * EMB, :])


def _relayout_tc(table_t, col0, n_cols):
    grid = -(-n_cols // TBLK)
    out_rows = -(-n_cols // PADW) * EMB
    off = col0 // TBLK
    eye = jnp.eye(EMB, dtype=jnp.float32)
    return pl.pallas_call(
        _transpose_body,
        grid=(grid,),
        in_specs=[
            pl.BlockSpec((EMB, TBLK), lambda i, o=off: (0, i + o)),
            pl.BlockSpec((EMB, EMB), lambda i: (0, 0)),
        ],
        out_specs=pl.BlockSpec((TBLK // 2, PADW), lambda i: (i, 0)),
        out_shape=jax.ShapeDtypeStruct((out_rows, PADW), jnp.float32),
        compiler_params=pltpu.CompilerParams(
            fuse_transposed_lhs_in_matmul=True),
    )(table_t, eye)


def _sc_relayout_body(tab_hbm, out_hbm, blk_v, outblk_v, buf_v):
    wid = lax.axis_index("s") * NC + lax.axis_index("c")
    cbase = wid * (SPLIT_M * PADW)
    iota = jax.lax.iota(jnp.int32, LANES)
    base_idx = iota * (LANES + 1)

    def chunk(k, carry):
        col0 = cbase + k * PADW
        pltpu.sync_copy(tab_hbm.at[:, pl.ds(col0, PADW)], blk_v)
        for ci in range(PADW // LANES):
            half = (ci * LANES) // EMB
            qb = (ci * LANES) % EMB
            for bi in range(EMB // LANES):
                for f in range(LANES):
                    buf_v[pl.ds(f * (LANES + 1), LANES)] = (
                        blk_v[bi * LANES + f, pl.ds(ci * LANES, LANES)])
                for c in range(LANES):
                    col = plsc.load_gather(buf_v, [base_idx + c])
                    outblk_v[qb + c,
                             pl.ds(half * EMB + bi * LANES, LANES)] = col
        pltpu.sync_copy(outblk_v,
                        out_hbm.at[pl.ds((cbase // PADW + k) * EMB, EMB), :])
        return carry

    lax.fori_loop(0, SPLIT_M, chunk, 0)


def _relayout_sc(table_t):
    mesh = plsc.VectorSubcoreMesh(core_axis_name="c", subcore_axis_name="s",
                                  num_cores=NC, num_subcores=NS)
    run = pl.kernel(
        _sc_relayout_body,
        out_type=jax.ShapeDtypeStruct((SPLIT // 2, PADW), jnp.float32),
        mesh=mesh,
        compiler_params=pltpu.CompilerParams(needs_layout_passes=False,
                                             use_tc_tiling_on_sc=True),
        scratch_types=[
            pltpu.VMEM((EMB, PADW), jnp.float32),
            pltpu.VMEM((EMB, PADW), jnp.float32),
            pltpu.VMEM((LANES * (LANES + 1),), jnp.float32),
        ],
    )
    return run(table_t)


def _body(user_hbm, movie_hbm, ulo_hbm, uhi_hbm, memb_hbm, ubias_hbm,
          mbias_hbm, gbias_hbm, out_hbm,
          uorig_v, morig_v, ulor_v, uhir_v, mpack_v, ubrow_v, mbrow_v,
          ulrows_v, uhrows_v, mrows_v, ubrows_v, mbrows_v, gb_v, out_v,
          buf_v, sem):
    wid = lax.axis_index("s") * NC + lax.axis_index("c")
    base = wid * BPW

    for j in range(NCHUNK):
        pltpu.sync_copy(user_hbm.at[pl.ds(base + j * CHUNK, CHUNK)],
                        uorig_v.at[j])
        pltpu.sync_copy(movie_hbm.at[pl.ds(base + j * CHUNK, CHUNK)],
                        morig_v.at[j])
    pltpu.sync_copy(gbias_hbm, gb_v.at[pl.ds(0, 1)])

    nhi_max = (1000000 - SPLIT) // PADW * EMB + EMB - 1
    for j in range(NCHUNK):
        for h in range(GROUPS):
            sl = pl.ds(h * LANES, LANES)
            u = uorig_v[j, sl]
            m = morig_v[j, sl]
            lane6 = lax.bitwise_and(u, 63)
            q = lax.bitwise_or(
                lax.shift_left(lax.shift_right_logical(u, 7), 6), lane6)
            uh = u - SPLIT
            qh = lax.bitwise_or(
                lax.shift_left(lax.shift_right_logical(uh, 7), 6), lane6)
            in_lo = u < SPLIT
            ulor_v[j, sl] = jnp.where(in_lo, q, 0)
            uhir_v[j, sl] = jnp.where(in_lo, 0, qh)
            mpack_v[j, sl] = lax.bitwise_or(
                lax.shift_left(lax.shift_right_logical(m, 7), 6),
                lax.bitwise_and(m, 63))
            ubrow_v[j, sl] = lax.shift_right_logical(u, 7)
            mbrow_v[j, sl] = lax.shift_right_logical(m, 7)

    gb = gb_v[pl.ds(0, LANES)][0]
    iota = jax.lax.iota(jnp.int32, LANES)
    base_idx = iota * (LANES + 1)

    def chunk_body(c, carry):
        c1 = pltpu.async_copy(ulo_hbm.at[ulor_v.at[c]], ulrows_v, sem)
        c2 = pltpu.async_copy(uhi_hbm.at[uhir_v.at[c]], uhrows_v, sem)
        c3 = pltpu.async_copy(memb_hbm.at[mpack_v.at[c]], mrows_v, sem)
        c4 = pltpu.async_copy(ubias_hbm.at[ubrow_v.at[c]], ubrows_v, sem)
        c5 = pltpu.async_copy(mbias_hbm.at[mbrow_v.at[c]], mbrows_v, sem)
        c1.wait()
        c2.wait()
        c3.wait()
        c4.wait()
        c5.wait()

        for g in range(GROUPS):
            rbase = g * LANES
            sl = pl.ds(rbase, LANES)
            uvec = uorig_v[c, sl]
            upar = lax.bitwise_and(lax.shift_right_logical(uvec, 6), 1)
            ulovec = (uvec < SPLIT).astype(jnp.int32)
            mpar = lax.bitwise_and(
                lax.shift_right_logical(morig_v[c, sl], 6), 1)
            for r in range(LANES):
                row = rbase + r
                pu = upar[r] != 0
                pm = mpar[r] != 0
                plo = ulovec[r] != 0
                acc = None
                for d in range(0, EMB, LANES):
                    ul0 = ulrows_v[row, pl.ds(d, LANES)]
                    ul1 = ulrows_v[row, pl.ds(EMB + d, LANES)]
                    uh0 = uhrows_v[row, pl.ds(d, LANES)]
                    uh1 = uhrows_v[row, pl.ds(EMB + d, LANES)]
                    m0 = mrows_v[row, pl.ds(d, LANES)]
                    m1 = mrows_v[row, pl.ds(EMB + d, LANES)]
                    uu = jnp.where(plo, jnp.where(pu, ul1, ul0),
                                   jnp.where(pu, uh1, uh0))
                    mm = jnp.where(pm, m1, m0)
                    p = uu * mm
                    acc = p if acc is None else acc + p
                buf_v[pl.ds(r * (LANES + 1), LANES)] = acc
            tot = None
            for col in range(LANES):
                v = plsc.load_gather(buf_v, [base_idx + col])
                tot = v if tot is None else tot + v
            ulane = lax.bitwise_and(uorig_v[c, sl], 127)
            mlane = lax.bitwise_and(morig_v[c, sl], 127)
            bu = plsc.load_gather(ubrows_v, [rbase + iota, ulane])
            bm = plsc.load_gather(mbrows_v, [rbase + iota, mlane])
            out_v[pl.ds(c * CHUNK + rbase, LANES)] = tot + bu + bm + gb
        return carry

    lax.fori_loop(0, NCHUNK, chunk_body, 0)

    pltpu.sync_copy(out_v, out_hbm.at[pl.ds(base, BPW)])


def kernel(user, movie, user_embedding, movie_embedding,
           user_bias_embedding, movie_bias_embedding, global_bias):
    n_user = user_embedding.shape[0]
    n_movie = movie_embedding.shape[0]
    ut = jnp.swapaxes(user_embedding, 0, 1)
    ulo = _relayout_sc(ut)
    uhi = _relayout_tc(ut, SPLIT, n_user - SPLIT)
    mpk = _relayout_tc(jnp.swapaxes(movie_embedding, 0, 1), 0, n_movie)
    ubr = -(-n_user // PADW)
    mbr = -(-n_movie // PADW)
    ubp = jnp.pad(user_bias_embedding,
                  ((0, ubr * PADW - n_user), (0, 0))).reshape(ubr, PADW)
    mbp = jnp.pad(movie_bias_embedding,
                  ((0, mbr * PADW - n_movie), (0, 0))).reshape(mbr, PADW)
    mesh = plsc.VectorSubcoreMesh(core_axis_name="c", subcore_axis_name="s",
                                  num_cores=NC, num_subcores=NS)
    run = pl.kernel(
        _body,
        out_type=jax.ShapeDtypeStruct((BATCH,), jnp.float32),
        mesh=mesh,
        compiler_params=pltpu.CompilerParams(needs_layout_passes=False,
                                             use_tc_tiling_on_sc=True),
        scratch_types=[
            pltpu.VMEM((NCHUNK, CHUNK), jnp.int32),
            pltpu.VMEM((NCHUNK, CHUNK), jnp.int32),
            pltpu.VMEM((NCHUNK, CHUNK), jnp.int32),
            pltpu.VMEM((NCHUNK, CHUNK), jnp.int32),
            pltpu.VMEM((NCHUNK, CHUNK), jnp.int32),
            pltpu.VMEM((NCHUNK, CHUNK), jnp.int32),
            pltpu.VMEM((NCHUNK, CHUNK), jnp.int32),
            pltpu.VMEM((CHUNK, PADW), jnp.float32),
            pltpu.VMEM((CHUNK, PADW), jnp.float32),
            pltpu.VMEM((CHUNK, PADW), jnp.float32),
            pltpu.VMEM((CHUNK, PADW), jnp.float32),
            pltpu.VMEM((CHUNK, PADW), jnp.float32),
            pltpu.VMEM((LANES,), jnp.float32),
            pltpu.VMEM((BPW,), jnp.float32),
            pltpu.VMEM((LANES * (LANES + 1),), jnp.float32),
            pltpu.SemaphoreType.DMA,
        ],
    )
    return run(user, movie, ulo, uhi, mpk, ubp, mbp, global_bias)


# speedup vs baseline: 1.3712x; 1.3712x over previous
import functools
import jax
import jax.numpy as jnp
from jax import lax
from jax.experimental import pallas as pl
from jax.experimental.pallas import tpu as pltpu
from jax.experimental.pallas import tpu_sc as plsc

NC = 2
NS = 16
NW = NC * NS
LANES = 16
BATCH = 16384
EMB = 64
PADW = 128
SUB = 8
BPW = BATCH // NW
CHUNK = 128
NCHUNK = BPW // CHUNK
GROUPS = CHUNK // LANES


def _body(user_hbm, movie_hbm, uemb_hbm, memb_hbm, ubias_hbm, mbias_hbm,
          gbias_hbm, out_hbm,
          uorig_v, morig_v, ublk_v, mblk_v, ubrow_v, mbrow_v,
          urows_v, mrows_v, ubrows_v, mbrows_v, gb_v, out_v, buf_v, sem):
    wid = lax.axis_index("s") * NC + lax.axis_index("c")
    base = wid * BPW

    for j in range(NCHUNK):
        pltpu.sync_copy(user_hbm.at[pl.ds(base + j * CHUNK, CHUNK)],
                        uorig_v.at[j])
        pltpu.sync_copy(movie_hbm.at[pl.ds(base + j * CHUNK, CHUNK)],
                        morig_v.at[j])
    pltpu.sync_copy(gbias_hbm, gb_v.at[pl.ds(0, 1)])

    for j in range(NCHUNK):
        for h in range(GROUPS):
            sl = pl.ds(h * LANES, LANES)
            u = uorig_v[j, sl]
            m = morig_v[j, sl]
            ublk_v[j, sl] = lax.shift_right_logical(u, 1)
            mblk_v[j, sl] = lax.shift_right_logical(m, 1)
            ubrow_v[j, sl] = lax.shift_right_logical(u, 7)
            mbrow_v[j, sl] = lax.shift_right_logical(m, 7)

    gb = gb_v[pl.ds(0, LANES)][0]
    iota = jax.lax.iota(jnp.int32, LANES)
    base_idx = iota * (LANES + 1)

    def chunk_body(c, carry):
        cu = pltpu.async_copy(uemb_hbm.at[ublk_v.at[c]], urows_v, sem)
        cm = pltpu.async_copy(memb_hbm.at[mblk_v.at[c]], mrows_v, sem)
        cub = pltpu.async_copy(ubias_hbm.at[ubrow_v.at[c]], ubrows_v, sem)
        cmb = pltpu.async_copy(mbias_hbm.at[mbrow_v.at[c]], mbrows_v, sem)
        cu.wait()
        cm.wait()
        cub.wait()
        cmb.wait()

        for g in range(GROUPS):
            rbase = g * LANES
            sl = pl.ds(rbase, LANES)
            upar = lax.bitwise_and(uorig_v[c, sl], 1)
            mpar = lax.bitwise_and(morig_v[c, sl], 1)
            for r in range(LANES):
                row = rbase + r
                pu = upar[r] != 0
                pm = mpar[r] != 0
                acc = None
                for d in range(0, EMB, LANES):
                    u0 = urows_v[row, pl.ds(d, LANES)]
                    u1 = urows_v[row, pl.ds(EMB + d, LANES)]
                    m0 = mrows_v[row, pl.ds(d, LANES)]
                    m1 = mrows_v[row, pl.ds(EMB + d, LANES)]
                    p = jnp.where(pu, u1, u0) * jnp.where(pm, m1, m0)
                    acc = p if acc is None else acc + p
                buf_v[pl.ds(r * (LANES + 1), LANES)] = acc
            tot = None
            for col in range(LANES):
                v = plsc.load_gather(buf_v, [base_idx + col])
                tot = v if tot is None else tot + v
            ulane = lax.bitwise_and(uorig_v[c, sl], 127)
            mlane = lax.bitwise_and(morig_v[c, sl], 127)
            bu = plsc.load_gather(ubrows_v, [rbase + iota, ulane])
            bm = plsc.load_gather(mbrows_v, [rbase + iota, mlane])
            out_v[pl.ds(c * CHUNK + rbase, LANES)] = tot + bu + bm + gb
        return carry

    lax.fori_loop(0, NCHUNK, chunk_body, 0)

    pltpu.sync_copy(out_v, out_hbm.at[pl.ds(base, BPW)])


def kernel(user, movie, user_embedding, movie_embedding,
           user_bias_embedding, movie_bias_embedding, global_bias):
    n_user = user_embedding.shape[0]
    n_movie = movie_embedding.shape[0]
    up3 = user_embedding.reshape(n_user // 2, PADW)
    mp3 = movie_embedding.reshape(n_movie // 2, PADW)
    ubr = -(-n_user // PADW)
    mbr = -(-n_movie // PADW)
    ubp = jnp.pad(user_bias_embedding,
                  ((0, ubr * PADW - n_user), (0, 0))).reshape(ubr, PADW)
    mbp = jnp.pad(movie_bias_embedding,
                  ((0, mbr * PADW - n_movie), (0, 0))).reshape(mbr, PADW)
    mesh = plsc.VectorSubcoreMesh(core_axis_name="c", subcore_axis_name="s",
                                  num_cores=NC, num_subcores=NS)
    run = pl.kernel(
        _body,
        out_type=jax.ShapeDtypeStruct((BATCH,), jnp.float32),
        mesh=mesh,
        compiler_params=pltpu.CompilerParams(needs_layout_passes=False,
                                             use_tc_tiling_on_sc=True),
        scratch_types=[
            pltpu.VMEM((NCHUNK, CHUNK), jnp.int32),
            pltpu.VMEM((NCHUNK, CHUNK), jnp.int32),
            pltpu.VMEM((NCHUNK, CHUNK), jnp.int32),
            pltpu.VMEM((NCHUNK, CHUNK), jnp.int32),
            pltpu.VMEM((NCHUNK, CHUNK), jnp.int32),
            pltpu.VMEM((NCHUNK, CHUNK), jnp.int32),
            pltpu.VMEM((CHUNK, PADW), jnp.float32),
            pltpu.VMEM((CHUNK, PADW), jnp.float32),
            pltpu.VMEM((CHUNK, PADW), jnp.float32),
            pltpu.VMEM((CHUNK, PADW), jnp.float32),
            pltpu.VMEM((LANES,), jnp.float32),
            pltpu.VMEM((BPW,), jnp.float32),
            pltpu.VMEM((LANES * (LANES + 1),), jnp.float32),
            pltpu.SemaphoreType.DMA,
        ],
    )
    return run(user, movie, up3, mp3, ubp, mbp, global_bias)


# speedup vs baseline: 1.8305x; 1.3349x over previous
import functools
import jax
import jax.numpy as jnp
from jax import lax
from jax.experimental import pallas as pl
from jax.experimental.pallas import tpu as pltpu
from jax.experimental.pallas import tpu_sc as plsc

NC = 2
NS = 16
NW = NC * NS
LANES = 16
BATCH = 16384
EMB = 64
PADW = 128
SUB = 8
BPW = BATCH // NW
CHUNK = 32
NCHUNK = BPW // CHUNK
GROUPS = CHUNK // LANES


def _body(user_hbm, movie_hbm, uemb_hbm, memb_hbm, ubias_hbm, mbias_hbm,
          gbias_hbm, out_hbm,
          uorig_v, morig_v, ubrow_v, mbrow_v, urows_v, mrows_v,
          ubrows_v, mbrows_v, gb_v, out_v, buf_v, sem, bsem):
    wid = lax.axis_index("s") * NC + lax.axis_index("c")
    base = wid * BPW

    for j in range(NCHUNK):
        pltpu.sync_copy(user_hbm.at[pl.ds(base + j * CHUNK, CHUNK)],
                        uorig_v.at[j])
        pltpu.sync_copy(movie_hbm.at[pl.ds(base + j * CHUNK, CHUNK)],
                        morig_v.at[j])
    pltpu.sync_copy(gbias_hbm, gb_v.at[pl.ds(0, 1)])

    for j in range(NCHUNK):
        for h in range(GROUPS):
            sl = pl.ds(h * LANES, LANES)
            ubrow_v[j, sl] = lax.shift_right_logical(uorig_v[j, sl], 7)
            mbrow_v[j, sl] = lax.shift_right_logical(morig_v[j, sl], 7)

    gb = gb_v[pl.ds(0, LANES)][0]
    iota = jax.lax.iota(jnp.int32, LANES)
    base_idx = iota * (LANES + 1)

    def chunk_body(c, carry):
        cub = pltpu.async_copy(ubias_hbm.at[ubrow_v.at[c]], ubrows_v, bsem)
        cmb = pltpu.async_copy(mbias_hbm.at[mbrow_v.at[c]], mbrows_v, bsem)
        copies = []
        for h in range(GROUPS):
            sl = pl.ds(h * LANES, LANES)
            ub8 = lax.shift_left(
                lax.shift_right_logical(uorig_v[c, sl], 3), 3)
            mb8 = lax.shift_left(
                lax.shift_right_logical(morig_v[c, sl], 3), 3)
            for j in range(LANES):
                row = h * LANES + j
                ru = pl.multiple_of(ub8[j], SUB)
                rm = pl.multiple_of(mb8[j], SUB)
                copies.append(pltpu.async_copy(
                    uemb_hbm.at[pl.ds(ru, SUB), :], urows_v.at[row], sem))
                copies.append(pltpu.async_copy(
                    memb_hbm.at[pl.ds(rm, SUB), :], mrows_v.at[row], sem))
        for cp in copies:
            cp.wait()
        cub.wait()
        cmb.wait()

        for g in range(GROUPS):
            rbase = g * LANES
            sl = pl.ds(rbase, LANES)
            usub = lax.bitwise_and(uorig_v[c, sl], SUB - 1)
            msub = lax.bitwise_and(morig_v[c, sl], SUB - 1)
            for r in range(LANES):
                row = rbase + r
                lu = usub[r]
                lm = msub[r]
                acc = None
                for d in range(0, EMB, LANES):
                    u = urows_v[row, lu, pl.ds(d, LANES)]
                    m = mrows_v[row, lm, pl.ds(d, LANES)]
                    p = u * m
                    acc = p if acc is None else acc + p
                buf_v[pl.ds(r * (LANES + 1), LANES)] = acc
            tot = None
            for col in range(LANES):
                v = plsc.load_gather(buf_v, [base_idx + col])
                tot = v if tot is None else tot + v
            ulane = lax.bitwise_and(uorig_v[c, sl], 127)
            mlane = lax.bitwise_and(morig_v[c, sl], 127)
            bu = plsc.load_gather(ubrows_v, [rbase + iota, ulane])
            bm = plsc.load_gather(mbrows_v, [rbase + iota, mlane])
            out_v[pl.ds(c * CHUNK + rbase, LANES)] = tot + bu + bm + gb
        return carry

    lax.fori_loop(0, NCHUNK, chunk_body, 0)

    pltpu.sync_copy(out_v, out_hbm.at[pl.ds(base, BPW)])


def kernel(user, movie, user_embedding, movie_embedding,
           user_bias_embedding, movie_bias_embedding, global_bias):
    n_user = user_embedding.shape[0]
    n_movie = movie_embedding.shape[0]
    ubr = -(-n_user // PADW)
    mbr = -(-n_movie // PADW)
    ubp = jnp.pad(user_bias_embedding,
                  ((0, ubr * PADW - n_user), (0, 0))).reshape(ubr, PADW)
    mbp = jnp.pad(movie_bias_embedding,
                  ((0, mbr * PADW - n_movie), (0, 0))).reshape(mbr, PADW)
    mesh = plsc.VectorSubcoreMesh(core_axis_name="c", subcore_axis_name="s",
                                  num_cores=NC, num_subcores=NS)
    run = pl.kernel(
        _body,
        out_type=jax.ShapeDtypeStruct((BATCH,), jnp.float32),
        mesh=mesh,
        compiler_params=pltpu.CompilerParams(needs_layout_passes=False,
                                             use_tc_tiling_on_sc=True),
        scratch_types=[
            pltpu.VMEM((NCHUNK, CHUNK), jnp.int32),
            pltpu.VMEM((NCHUNK, CHUNK), jnp.int32),
            pltpu.VMEM((NCHUNK, CHUNK), jnp.int32),
            pltpu.VMEM((NCHUNK, CHUNK), jnp.int32),
            pltpu.VMEM((CHUNK, SUB, EMB), jnp.float32),
            pltpu.VMEM((CHUNK, SUB, EMB), jnp.float32),
            pltpu.VMEM((CHUNK, PADW), jnp.float32),
            pltpu.VMEM((CHUNK, PADW), jnp.float32),
            pltpu.VMEM((LANES,), jnp.float32),
            pltpu.VMEM((BPW,), jnp.float32),
            pltpu.VMEM((LANES * (LANES + 1),), jnp.float32),
            pltpu.SemaphoreType.DMA,
            pltpu.SemaphoreType.DMA,
        ],
    )
    return run(user, movie, user_embedding, movie_embedding, ubp, mbp,
               global_bias)


# speedup vs baseline: 1.8708x; 1.0220x over previous
import functools
import jax
import jax.numpy as jnp
from jax import lax
from jax.experimental import pallas as pl
from jax.experimental.pallas import tpu as pltpu
from jax.experimental.pallas import tpu_sc as plsc

NC = 2
NS = 16
NW = NC * NS
LANES = 16
BATCH = 16384
EMB = 64
PADW = 128
SUB = 8
BPW = BATCH // NW
CHUNK = 16
NCHUNK = BPW // CHUNK


def _body(user_hbm, movie_hbm, uemb_hbm, memb_hbm, ubias_hbm, mbias_hbm,
          gbias_hbm, out_hbm,
          uorig_v, morig_v, ubrow_v, mbrow_v, urows_v, mrows_v,
          ubrows_v, mbrows_v, gb_v, out_v, buf_v, sem, bsem):
    wid = lax.axis_index("s") * NC + lax.axis_index("c")
    base = wid * BPW

    for j in range(0, BPW, PADW):
        pltpu.sync_copy(user_hbm.at[pl.ds(base + j, PADW)],
                        uorig_v.at[pl.ds(j, PADW)])
        pltpu.sync_copy(movie_hbm.at[pl.ds(base + j, PADW)],
                        morig_v.at[pl.ds(j, PADW)])
    pltpu.sync_copy(gbias_hbm, gb_v.at[pl.ds(0, 1)])

    for j in range(0, BPW, LANES):
        sl = pl.ds(j, LANES)
        ubrow_v[sl] = lax.shift_right_logical(uorig_v[sl], 7)
        mbrow_v[sl] = lax.shift_right_logical(morig_v[sl], 7)

    gb = gb_v[pl.ds(0, LANES)][0]
    iota = jax.lax.iota(jnp.int32, LANES)
    base_idx = iota * (LANES + 1)

    def fire(c, slot):
        ebase = c * CHUNK
        uvec = uorig_v[pl.ds(ebase, LANES)]
        mvec = morig_v[pl.ds(ebase, LANES)]
        ub8 = lax.shift_left(lax.shift_right_logical(uvec, 3), 3)
        mb8 = lax.shift_left(lax.shift_right_logical(mvec, 3), 3)
        handles = [
            pltpu.async_copy(ubias_hbm.at[ubrow_v.at[pl.ds(ebase, CHUNK)]],
                             ubrows_v.at[pl.ds(slot * CHUNK, CHUNK)], bsem),
            pltpu.async_copy(mbias_hbm.at[mbrow_v.at[pl.ds(ebase, CHUNK)]],
                             mbrows_v.at[pl.ds(slot * CHUNK, CHUNK)], bsem),
        ]
        for j in range(LANES):
            ru = pl.multiple_of(ub8[j], SUB)
            rm = pl.multiple_of(mb8[j], SUB)
            handles.append(pltpu.async_copy(
                uemb_hbm.at[pl.ds(ru, SUB), :], urows_v.at[slot, j], sem))
            handles.append(pltpu.async_copy(
                memb_hbm.at[pl.ds(rm, SUB), :], mrows_v.at[slot, j], sem))
        return handles

    def compute(c, slot):
        ebase = c * CHUNK
        uvec = uorig_v[pl.ds(ebase, LANES)]
        mvec = morig_v[pl.ds(ebase, LANES)]
        usub = lax.bitwise_and(uvec, SUB - 1)
        msub = lax.bitwise_and(mvec, SUB - 1)
        for r in range(LANES):
            lu = usub[r]
            lm = msub[r]
            acc = None
            for d in range(0, EMB, LANES):
                u = urows_v[slot, r, lu, pl.ds(d, LANES)]
                m = mrows_v[slot, r, lm, pl.ds(d, LANES)]
                p = u * m
                acc = p if acc is None else acc + p
            buf_v[pl.ds(r * (LANES + 1), LANES)] = acc
        tot = None
        for col in range(LANES):
            v = plsc.load_gather(buf_v, [base_idx + col])
            tot = v if tot is None else tot + v
        ulane = lax.bitwise_and(uvec, 127)
        mlane = lax.bitwise_and(mvec, 127)
        bu = plsc.load_gather(ubrows_v, [iota + slot * CHUNK, ulane])
        bm = plsc.load_gather(mbrows_v, [iota + slot * CHUNK, mlane])
        out_v[pl.ds(ebase, LANES)] = tot + bu + bm + gb

    def pair_body(p, carry):
        c0 = p * 2
        c1 = c0 + 1
        h0 = fire(c0, 0)
        h1 = fire(c1, 1)
        for h in h0:
            h.wait()
        compute(c0, 0)
        for h in h1:
            h.wait()
        compute(c1, 1)
        return carry

    lax.fori_loop(0, NCHUNK // 2, pair_body, 0)

    pltpu.sync_copy(out_v, out_hbm.at[pl.ds(base, BPW)])


def kernel(user, movie, user_embedding, movie_embedding,
           user_bias_embedding, movie_bias_embedding, global_bias):
    n_user = user_embedding.shape[0]
    n_movie = movie_embedding.shape[0]
    ubr = -(-n_user // PADW)
    mbr = -(-n_movie // PADW)
    ubp = jnp.pad(user_bias_embedding,
                  ((0, ubr * PADW - n_user), (0, 0))).reshape(ubr, PADW)
    mbp = jnp.pad(movie_bias_embedding,
                  ((0, mbr * PADW - n_movie), (0, 0))).reshape(mbr, PADW)
    mesh = plsc.VectorSubcoreMesh(core_axis_name="c", subcore_axis_name="s",
                                  num_cores=NC, num_subcores=NS)
    run = pl.kernel(
        _body,
        out_type=jax.ShapeDtypeStruct((BATCH,), jnp.float32),
        mesh=mesh,
        compiler_params=pltpu.CompilerParams(needs_layout_passes=False,
                                             use_tc_tiling_on_sc=True),
        scratch_types=[
            pltpu.VMEM((BPW,), jnp.int32),
            pltpu.VMEM((BPW,), jnp.int32),
            pltpu.VMEM((BPW,), jnp.int32),
            pltpu.VMEM((BPW,), jnp.int32),
            pltpu.VMEM((2, CHUNK, SUB, EMB), jnp.float32),
            pltpu.VMEM((2, CHUNK, SUB, EMB), jnp.float32),
            pltpu.VMEM((2 * CHUNK, PADW), jnp.float32),
            pltpu.VMEM((2 * CHUNK, PADW), jnp.float32),
            pltpu.VMEM((LANES,), jnp.float32),
            pltpu.VMEM((BPW,), jnp.float32),
            pltpu.VMEM((LANES * (LANES + 1),), jnp.float32),
            pltpu.SemaphoreType.DMA,
            pltpu.SemaphoreType.DMA,
        ],
    )
    return run(user, movie, user_embedding, movie_embedding, ubp, mbp,
               global_bias)
